# baseline (speedup 1.0000x reference)
import functools

import jax
import jax.numpy as jnp
from jax import lax
from jax.experimental import pallas as pl
from jax.experimental.pallas import tpu as pltpu
from jax.experimental.pallas import tpu_sc as plsc

N = 10000
N_PAD = 10240
E = 320000
D_IN = 128
D_OUT = 128
DH = D_IN // 2

NC = 2
NS = 16
K = 80
NB = 250
EPT_REAL = E // NS
NBUF = 5
ROWS_PER_TILE = N_PAD // NS
ZROWS = 32


def _sc_body(row_hbm, col_hbm, x_hbm, out_hbm,
             row_v, col_v, rows0, rows1, rows2, rows3, rows4, zbuf,
             acc, sg0, sg1, sg2, sg3, sg4, ss0, ss1, ss2, ss3, ss4):
    c = lax.axis_index("c")
    s = lax.axis_index("s")

    zeros = jnp.zeros((16,), jnp.float32)

    def _zero(i, carry):
        for j in range(DH // 16):
            zbuf[i, pl.ds(j * 16, 16)] = zeros
        return carry

    lax.fori_loop(0, ZROWS, _zero, 0)
    for kk in range(ROWS_PER_TILE // ZROWS):
        pltpu.sync_copy(zbuf, acc.at[pl.ds(s * ROWS_PER_TILE + kk * ZROWS, ZROWS)])
    plsc.subcore_barrier()

    pltpu.sync_copy(row_hbm.at[s], row_v)
    pltpu.sync_copy(col_hbm.at[s], col_v)

    rows = (rows0, rows1, rows2, rows3, rows4)
    sg = (sg0, sg1, sg2, sg3, sg4)
    ss = (ss0, ss1, ss2, ss3, ss4)

    def _xform(j):
        for k in range(K // 16):
            v = col_v[j, pl.ds(16 * k, 16)]
            col_v[j, pl.ds(16 * k, 16)] = v + v + c

    def _gather(j, b):
        pltpu.async_copy(x_hbm.at[col_v.at[j]], rows[b], sg[b])

    def _gather_wait(b):
        pltpu.make_async_copy(x_hbm.at[col_v.at[0]], rows[b], sg[b]).wait()

    def _scatter(j, b):
        pltpu.async_copy(rows[b], acc.at[row_v.at[j]], ss[b], add=True)

    def _scatter_wait(b):
        pltpu.make_async_copy(x_hbm.at[col_v.at[0]], rows[b], ss[b]).wait()

    _xform(0)
    _gather(0, 0)
    _xform(1)
    _gather(1, 1)

    def _group(g, carry):
        for b in range(NBUF):
            j = NBUF * g + b
            nb = (b + 2) % NBUF
            if b < 3:
                @pl.when(g > 0)
                def _():
                    _scatter_wait(nb)
                _xform(j + 2)
                _gather(j + 2, nb)
            else:
                _scatter_wait(nb)
                @pl.when(g < NB // NBUF - 1)
                def _():
                    _xform(j + 2)
                    _gather(j + 2, nb)
            _gather_wait(b)
            _scatter(j, b)
        return carry

    lax.fori_loop(0, NB // NBUF, _group, 0)
    _scatter_wait(2)
    _scatter_wait(3)
    _scatter_wait(4)
    plsc.subcore_barrier()

    for kk in range(ROWS_PER_TILE // ZROWS):
        sl = pl.ds(s * ROWS_PER_TILE + kk * ZROWS, ZROWS)
        pltpu.sync_copy(acc.at[sl], zbuf)
        pltpu.sync_copy(zbuf, out_hbm.at[sl, pl.ds(c * DH, DH)])


_sc_segment_sum = functools.partial(
    pl.kernel,
    out_type=jax.ShapeDtypeStruct((N_PAD, D_IN), jnp.float32),
    mesh=plsc.VectorSubcoreMesh(core_axis_name="c", subcore_axis_name="s"),
    compiler_params=pltpu.CompilerParams(use_tc_tiling_on_sc=False),
    scratch_types=[
        pltpu.VMEM((NB, K), jnp.int32),
        pltpu.VMEM((NB, K), jnp.int32),
        pltpu.VMEM((K, DH), jnp.float32),
        pltpu.VMEM((K, DH), jnp.float32),
        pltpu.VMEM((K, DH), jnp.float32),
        pltpu.VMEM((K, DH), jnp.float32),
        pltpu.VMEM((K, DH), jnp.float32),
        pltpu.VMEM((ZROWS, DH), jnp.float32),
        pltpu.VMEM_SHARED((N_PAD, DH), jnp.float32),
        pltpu.SemaphoreType.DMA,
        pltpu.SemaphoreType.DMA,
        pltpu.SemaphoreType.DMA,
        pltpu.SemaphoreType.DMA,
        pltpu.SemaphoreType.DMA,
        pltpu.SemaphoreType.DMA,
        pltpu.SemaphoreType.DMA,
        pltpu.SemaphoreType.DMA,
        pltpu.SemaphoreType.DMA,
        pltpu.SemaphoreType.DMA,
    ],
)(_sc_body)


def _mm_body(p_ref, w_ref, o_ref):
    o_ref[...] = jnp.maximum(
        jnp.dot(p_ref[...], w_ref[...], preferred_element_type=jnp.float32), 0.0
    )


_MM_BLOCK = 1000


def _mm_relu(p, w):
    return pl.pallas_call(
        _mm_body,
        grid=(N // _MM_BLOCK,),
        in_specs=[
            pl.BlockSpec((_MM_BLOCK, D_IN), lambda i: (i, 0)),
            pl.BlockSpec((D_IN, D_OUT), lambda i: (0, 0)),
        ],
        out_specs=pl.BlockSpec((_MM_BLOCK, D_OUT), lambda i: (i, 0)),
        out_shape=jax.ShapeDtypeStruct((N, D_OUT), jnp.float32),
    )(p, w)


def kernel(adj_edge_index, inputs, W):
    row_p = adj_edge_index[0].reshape(NS, NB, K)
    col_p = adj_edge_index[1].reshape(NS, NB, K)
    x_r = inputs.reshape(2 * N, DH)
    agg = _sc_segment_sum(row_p, col_p, x_r)
    return _mm_relu(agg, W)


# speedup vs baseline: 1.0529x; 1.0529x over previous
import functools

import jax
import jax.numpy as jnp
from jax import lax
from jax.experimental import pallas as pl
from jax.experimental.pallas import tpu as pltpu
from jax.experimental.pallas import tpu_sc as plsc

N = 10000
N_PAD = 10240
E = 320000
D_IN = 128
D_OUT = 128
DH = D_IN // 2

NC = 2
NS = 16
K = 128
NB = 160
EPT = NB * K
EPT_REAL = E // NS
PAD = EPT - EPT_REAL
NBUF = 5
ROWS_PER_TILE = N_PAD // NS
ZROWS = 64


def _sc_body(row_hbm, col_hbm, x_hbm, out_hbm,
             row_v, col_v, rows0, rows1, rows2, rows3, rows4, zbuf,
             acc, sg0, sg1, sg2, sg3, sg4, ss0, ss1, ss2, ss3, ss4):
    c = lax.axis_index("c")
    s = lax.axis_index("s")

    zeros = jnp.zeros((16,), jnp.float32)

    def _zero(i, carry):
        for j in range(DH // 16):
            zbuf[i, pl.ds(j * 16, 16)] = zeros
        return carry

    lax.fori_loop(0, ZROWS, _zero, 0)
    for kk in range(ROWS_PER_TILE // ZROWS):
        pltpu.sync_copy(zbuf, acc.at[pl.ds(s * ROWS_PER_TILE + kk * ZROWS, ZROWS)])
    plsc.subcore_barrier()

    pltpu.sync_copy(row_hbm.at[s], row_v)
    pltpu.sync_copy(col_hbm.at[s], col_v)

    rows = (rows0, rows1, rows2, rows3, rows4)
    sg = (sg0, sg1, sg2, sg3, sg4)
    ss = (ss0, ss1, ss2, ss3, ss4)

    def _xform(j):
        for k in range(K // 16):
            v = col_v[j, pl.ds(16 * k, 16)]
            col_v[j, pl.ds(16 * k, 16)] = v + v + c

    def _gather(j, b):
        pltpu.async_copy(x_hbm.at[col_v.at[j]], rows[b], sg[b])

    def _gather_wait(b):
        pltpu.make_async_copy(x_hbm.at[col_v.at[0]], rows[b], sg[b]).wait()

    def _scatter(j, b):
        pltpu.async_copy(rows[b], acc.at[row_v.at[j]], ss[b], add=True)

    def _scatter_wait(b):
        pltpu.make_async_copy(x_hbm.at[col_v.at[0]], rows[b], ss[b]).wait()

    _xform(0)
    _gather(0, 0)
    _xform(1)
    _gather(1, 1)

    def _group(g, carry):
        for b in range(NBUF):
            j = NBUF * g + b
            nb = (b + 2) % NBUF
            if b < 3:
                @pl.when(g > 0)
                def _():
                    _scatter_wait(nb)
                _xform(j + 2)
                _gather(j + 2, nb)
            else:
                _scatter_wait(nb)
                @pl.when(g < NB // NBUF - 1)
                def _():
                    _xform(j + 2)
                    _gather(j + 2, nb)
            _gather_wait(b)
            _scatter(j, b)
        return carry

    lax.fori_loop(0, NB // NBUF, _group, 0)
    _scatter_wait(2)
    _scatter_wait(3)
    _scatter_wait(4)
    plsc.subcore_barrier()

    for kk in range(ROWS_PER_TILE // ZROWS):
        sl = pl.ds(s * ROWS_PER_TILE + kk * ZROWS, ZROWS)
        pltpu.sync_copy(acc.at[sl], zbuf)
        pltpu.sync_copy(zbuf, out_hbm.at[sl, pl.ds(c * DH, DH)])


_sc_segment_sum = functools.partial(
    pl.kernel,
    out_type=jax.ShapeDtypeStruct((N_PAD, D_IN), jnp.float32),
    mesh=plsc.VectorSubcoreMesh(core_axis_name="c", subcore_axis_name="s"),
    compiler_params=pltpu.CompilerParams(use_tc_tiling_on_sc=False),
    scratch_types=[
        pltpu.VMEM((NB, K), jnp.int32),
        pltpu.VMEM((NB, K), jnp.int32),
        pltpu.VMEM((K, DH), jnp.float32),
        pltpu.VMEM((K, DH), jnp.float32),
        pltpu.VMEM((K, DH), jnp.float32),
        pltpu.VMEM((K, DH), jnp.float32),
        pltpu.VMEM((K, DH), jnp.float32),
        pltpu.VMEM((ZROWS, DH), jnp.float32),
        pltpu.VMEM_SHARED((N_PAD, DH), jnp.float32),
        pltpu.SemaphoreType.DMA,
        pltpu.SemaphoreType.DMA,
        pltpu.SemaphoreType.DMA,
        pltpu.SemaphoreType.DMA,
        pltpu.SemaphoreType.DMA,
        pltpu.SemaphoreType.DMA,
        pltpu.SemaphoreType.DMA,
        pltpu.SemaphoreType.DMA,
        pltpu.SemaphoreType.DMA,
        pltpu.SemaphoreType.DMA,
    ],
)(_sc_body)


def _mm_body(p_ref, w_ref, o_ref):
    o_ref[...] = jnp.maximum(
        jnp.dot(p_ref[...], w_ref[...], preferred_element_type=jnp.float32), 0.0
    )


_MM_BLOCK = 2000


def _mm_relu(p, w):
    return pl.pallas_call(
        _mm_body,
        grid=(N // _MM_BLOCK,),
        in_specs=[
            pl.BlockSpec((_MM_BLOCK, D_IN), lambda i: (i, 0)),
            pl.BlockSpec((D_IN, D_OUT), lambda i: (0, 0)),
        ],
        out_specs=pl.BlockSpec((_MM_BLOCK, D_OUT), lambda i: (i, 0)),
        out_shape=jax.ShapeDtypeStruct((N, D_OUT), jnp.float32),
    )(p, w)


def kernel(adj_edge_index, inputs, W):
    row_t = adj_edge_index[0].reshape(NS, EPT_REAL)
    col_t = adj_edge_index[1].reshape(NS, EPT_REAL)
    pad_r = N + (jnp.arange(PAD, dtype=jnp.int32) % (N_PAD - N))
    pad_c = (jnp.arange(PAD, dtype=jnp.int32) * 41) % N
    row_p = jnp.concatenate(
        [row_t, jnp.broadcast_to(pad_r, (NS, PAD))], axis=1).reshape(NS, NB, K)
    col_p = jnp.concatenate(
        [col_t, jnp.broadcast_to(pad_c, (NS, PAD))], axis=1).reshape(NS, NB, K)
    x_r = inputs.reshape(2 * N, DH)
    agg = _sc_segment_sum(row_p, col_p, x_r)
    return _mm_relu(agg, W)


# speedup vs baseline: 1.0883x; 1.0337x over previous
import functools

import jax
import jax.numpy as jnp
from jax import lax
from jax.experimental import pallas as pl
from jax.experimental.pallas import tpu as pltpu
from jax.experimental.pallas import tpu_sc as plsc

N = 10000
N_PAD = 10240
E = 320000
D_IN = 128
D_OUT = 128
DH = D_IN // 2

NC = 2
NS = 16
K = 128
NB = 160
EPT = NB * K
EPT_REAL = E // NS
PAD = EPT - EPT_REAL
NBUF = 5
ROWS_PER_TILE = N_PAD // NS
ZROWS = 64


def _sc_body(row_hbm, col_hbm, x_hbm, out_hbm,
             row_v, col_v, rows0, rows1, rows2, rows3, rows4,
             acc, sg0, sg1, sg2, sg3, sg4, ss0, ss1, ss2, ss3, ss4):
    c = lax.axis_index("c")
    s = lax.axis_index("s")

    rows = (rows0, rows1, rows2, rows3, rows4)
    sg = (sg0, sg1, sg2, sg3, sg4)
    ss = (ss0, ss1, ss2, ss3, ss4)

    pltpu.async_copy(row_hbm.at[s], row_v, ss0)
    pltpu.async_copy(col_hbm.at[s], col_v, ss1)

    zeros = jnp.zeros((16,), jnp.float32)

    def _zero(i, carry):
        for j in range(DH // 16):
            rows0[i, pl.ds(j * 16, 16)] = zeros
        return carry

    lax.fori_loop(0, K, _zero, 0)
    NZ = ROWS_PER_TILE // K
    for kk in range(NZ):
        pltpu.async_copy(rows0, acc.at[pl.ds(s * ROWS_PER_TILE + kk * K, K)], sg0)
    for kk in range(NZ):
        pltpu.make_async_copy(rows0, acc.at[pl.ds(s * ROWS_PER_TILE, K)], sg0).wait()
    pltpu.make_async_copy(row_hbm.at[s], row_v, ss0).wait()
    pltpu.make_async_copy(col_hbm.at[s], col_v, ss1).wait()
    plsc.subcore_barrier()

    def _xform(j):
        for k in range(K // 16):
            v = col_v[j, pl.ds(16 * k, 16)]
            col_v[j, pl.ds(16 * k, 16)] = v + v + c

    def _gather(j, b):
        pltpu.async_copy(x_hbm.at[col_v.at[j]], rows[b], sg[b])

    def _gather_wait(b):
        pltpu.make_async_copy(x_hbm.at[col_v.at[0]], rows[b], sg[b]).wait()

    def _scatter(j, b):
        pltpu.async_copy(rows[b], acc.at[row_v.at[j]], ss[b], add=True)

    def _scatter_wait(b):
        pltpu.make_async_copy(x_hbm.at[col_v.at[0]], rows[b], ss[b]).wait()

    _xform(0)
    _gather(0, 0)
    _xform(1)
    _gather(1, 1)

    def _group(g, carry):
        for b in range(NBUF):
            j = NBUF * g + b
            nb = (b + 2) % NBUF
            if b < 3:
                @pl.when(g > 0)
                def _():
                    _scatter_wait(nb)
                _xform(j + 2)
                _gather(j + 2, nb)
            else:
                _scatter_wait(nb)
                @pl.when(g < NB // NBUF - 1)
                def _():
                    _xform(j + 2)
                    _gather(j + 2, nb)
            _gather_wait(b)
            _scatter(j, b)
        return carry

    lax.fori_loop(0, NB // NBUF, _group, 0)
    _scatter_wait(2)
    _scatter_wait(3)
    _scatter_wait(4)
    plsc.subcore_barrier()

    NW = ROWS_PER_TILE // K

    def _wsl(kk):
        return pl.ds(s * ROWS_PER_TILE + kk * K, K)

    pltpu.async_copy(acc.at[_wsl(0)], rows[0], sg[0])
    for kk in range(NW):
        b = kk % 2
        pltpu.make_async_copy(acc.at[_wsl(kk)], rows[b], sg[b]).wait()
        if kk >= 1:
            pltpu.make_async_copy(rows[1 - b], out_hbm.at[_wsl(kk - 1), pl.ds(c * DH, DH)], ss[1 - b]).wait()
        if kk + 1 < NW:
            pltpu.async_copy(acc.at[_wsl(kk + 1)], rows[1 - b], sg[1 - b])
        pltpu.async_copy(rows[b], out_hbm.at[_wsl(kk), pl.ds(c * DH, DH)], ss[b])
    pltpu.make_async_copy(rows[(NW - 1) % 2], out_hbm.at[_wsl(NW - 1), pl.ds(c * DH, DH)], ss[(NW - 1) % 2]).wait()


_sc_segment_sum = functools.partial(
    pl.kernel,
    out_type=jax.ShapeDtypeStruct((N_PAD, D_IN), jnp.float32),
    mesh=plsc.VectorSubcoreMesh(core_axis_name="c", subcore_axis_name="s"),
    compiler_params=pltpu.CompilerParams(use_tc_tiling_on_sc=False),
    scratch_types=[
        pltpu.VMEM((NB, K), jnp.int32),
        pltpu.VMEM((NB, K), jnp.int32),
        pltpu.VMEM((K, DH), jnp.float32),
        pltpu.VMEM((K, DH), jnp.float32),
        pltpu.VMEM((K, DH), jnp.float32),
        pltpu.VMEM((K, DH), jnp.float32),
        pltpu.VMEM((K, DH), jnp.float32),
        pltpu.VMEM_SHARED((N_PAD, DH), jnp.float32),
        pltpu.SemaphoreType.DMA,
        pltpu.SemaphoreType.DMA,
        pltpu.SemaphoreType.DMA,
        pltpu.SemaphoreType.DMA,
        pltpu.SemaphoreType.DMA,
        pltpu.SemaphoreType.DMA,
        pltpu.SemaphoreType.DMA,
        pltpu.SemaphoreType.DMA,
        pltpu.SemaphoreType.DMA,
        pltpu.SemaphoreType.DMA,
    ],
)(_sc_body)


def _mm_body(p_ref, w_ref, o_ref):
    o_ref[...] = jnp.maximum(
        jnp.dot(p_ref[...], w_ref[...], preferred_element_type=jnp.float32), 0.0
    )


_MM_BLOCK = 2000


def _mm_relu(p, w):
    return pl.pallas_call(
        _mm_body,
        grid=(N // _MM_BLOCK,),
        in_specs=[
            pl.BlockSpec((_MM_BLOCK, D_IN), lambda i: (i, 0)),
            pl.BlockSpec((D_IN, D_OUT), lambda i: (0, 0)),
        ],
        out_specs=pl.BlockSpec((_MM_BLOCK, D_OUT), lambda i: (i, 0)),
        out_shape=jax.ShapeDtypeStruct((N, D_OUT), jnp.float32),
    )(p, w)


def kernel(adj_edge_index, inputs, W):
    row_t = adj_edge_index[0].reshape(NS, EPT_REAL)
    col_t = adj_edge_index[1].reshape(NS, EPT_REAL)
    pad_r = N + (jnp.arange(PAD, dtype=jnp.int32) % (N_PAD - N))
    pad_c = (jnp.arange(PAD, dtype=jnp.int32) * 41) % N
    row_p = jnp.concatenate(
        [row_t, jnp.broadcast_to(pad_r, (NS, PAD))], axis=1).reshape(NS, NB, K)
    col_p = jnp.concatenate(
        [col_t, jnp.broadcast_to(pad_c, (NS, PAD))], axis=1).reshape(NS, NB, K)
    x_r = inputs.reshape(2 * N, DH)
    agg = _sc_segment_sum(row_p, col_p, x_r)
    return _mm_relu(agg, W)


# speedup vs baseline: 1.0932x; 1.0045x over previous
import functools

import jax
import jax.numpy as jnp
from jax import lax
from jax.experimental import pallas as pl
from jax.experimental.pallas import tpu as pltpu
from jax.experimental.pallas import tpu_sc as plsc

N = 10000
N_PAD = 10240
E = 320000
D_IN = 128
D_OUT = 128
DH = D_IN // 2

NC = 2
NS = 16
K = 128
NB = 160
EPT = NB * K
EPT_REAL = E // NS
PAD = EPT - EPT_REAL
NBUF = 5
ROWS_PER_TILE = N_PAD // NS
ZROWS = 64


def _sc_body(row_hbm, col_hbm, x_hbm, out_hbm,
             row_v, col_v, rows0, rows1, rows2, rows3, rows4,
             acc, sg0, sg1, sg2, sg3, sg4, ss0, ss1, ss2, ss3, ss4):
    c = lax.axis_index("c")
    s = lax.axis_index("s")

    rows = (rows0, rows1, rows2, rows3, rows4)
    sg = (sg0, sg1, sg2, sg3, sg4)
    ss = (ss0, ss1, ss2, ss3, ss4)

    pltpu.async_copy(row_hbm.at[s], row_v, ss0)
    pltpu.async_copy(col_hbm.at[s], col_v, ss1)

    zeros = jnp.zeros((16,), jnp.float32)

    def _zero(i, carry):
        for j in range(DH // 16):
            rows0[i, pl.ds(j * 16, 16)] = zeros
        return carry

    lax.fori_loop(0, K, _zero, 0)
    NZ = ROWS_PER_TILE // K
    for kk in range(NZ):
        pltpu.async_copy(rows0, acc.at[pl.ds(s * ROWS_PER_TILE + kk * K, K)], sg0)
    for kk in range(NZ):
        pltpu.make_async_copy(rows0, acc.at[pl.ds(s * ROWS_PER_TILE, K)], sg0).wait()
    pltpu.make_async_copy(row_hbm.at[s], row_v, ss0).wait()
    pltpu.make_async_copy(col_hbm.at[s], col_v, ss1).wait()

    def _xform(j):
        for k in range(K // 16):
            v = col_v[j, pl.ds(16 * k, 16)]
            col_v[j, pl.ds(16 * k, 16)] = v + v + c

    def _gather(j, b):
        pltpu.async_copy(x_hbm.at[col_v.at[j]], rows[b], sg[b])

    def _gather_wait(b):
        pltpu.make_async_copy(x_hbm.at[col_v.at[0]], rows[b], sg[b]).wait()

    def _scatter(j, b):
        pltpu.async_copy(rows[b], acc.at[row_v.at[j]], ss[b], add=True)

    def _scatter_wait(b):
        pltpu.make_async_copy(x_hbm.at[col_v.at[0]], rows[b], ss[b]).wait()

    _xform(0)
    _gather(0, 0)
    _xform(1)
    _gather(1, 1)
    plsc.subcore_barrier()

    def _group(g, carry):
        for b in range(NBUF):
            j = NBUF * g + b
            nb = (b + 2) % NBUF
            if b < 3:
                @pl.when(g > 0)
                def _():
                    _scatter_wait(nb)
                _xform(j + 2)
                _gather(j + 2, nb)
            else:
                _scatter_wait(nb)
                @pl.when(g < NB // NBUF - 1)
                def _():
                    _xform(j + 2)
                    _gather(j + 2, nb)
            _gather_wait(b)
            _scatter(j, b)
        return carry

    lax.fori_loop(0, NB // NBUF, _group, 0)
    _scatter_wait(2)
    _scatter_wait(3)
    _scatter_wait(4)
    plsc.subcore_barrier()

    NW = ROWS_PER_TILE // K

    def _wsl(kk):
        return pl.ds(s * ROWS_PER_TILE + kk * K, K)

    pltpu.async_copy(acc.at[_wsl(0)], rows[0], sg[0])
    for kk in range(NW):
        b = kk % 2
        pltpu.make_async_copy(acc.at[_wsl(kk)], rows[b], sg[b]).wait()
        if kk >= 1:
            pltpu.make_async_copy(rows[1 - b], out_hbm.at[_wsl(kk - 1), pl.ds(c * DH, DH)], ss[1 - b]).wait()
        if kk + 1 < NW:
            pltpu.async_copy(acc.at[_wsl(kk + 1)], rows[1 - b], sg[1 - b])
        pltpu.async_copy(rows[b], out_hbm.at[_wsl(kk), pl.ds(c * DH, DH)], ss[b])
    pltpu.make_async_copy(rows[(NW - 1) % 2], out_hbm.at[_wsl(NW - 1), pl.ds(c * DH, DH)], ss[(NW - 1) % 2]).wait()


_sc_segment_sum = functools.partial(
    pl.kernel,
    out_type=jax.ShapeDtypeStruct((N_PAD, D_IN), jnp.float32),
    mesh=plsc.VectorSubcoreMesh(core_axis_name="c", subcore_axis_name="s"),
    compiler_params=pltpu.CompilerParams(use_tc_tiling_on_sc=False),
    scratch_types=[
        pltpu.VMEM((NB, K), jnp.int32),
        pltpu.VMEM((NB, K), jnp.int32),
        pltpu.VMEM((K, DH), jnp.float32),
        pltpu.VMEM((K, DH), jnp.float32),
        pltpu.VMEM((K, DH), jnp.float32),
        pltpu.VMEM((K, DH), jnp.float32),
        pltpu.VMEM((K, DH), jnp.float32),
        pltpu.VMEM_SHARED((N_PAD, DH), jnp.float32),
        pltpu.SemaphoreType.DMA,
        pltpu.SemaphoreType.DMA,
        pltpu.SemaphoreType.DMA,
        pltpu.SemaphoreType.DMA,
        pltpu.SemaphoreType.DMA,
        pltpu.SemaphoreType.DMA,
        pltpu.SemaphoreType.DMA,
        pltpu.SemaphoreType.DMA,
        pltpu.SemaphoreType.DMA,
        pltpu.SemaphoreType.DMA,
    ],
)(_sc_body)


def _mm_body(p_ref, w_ref, o_ref):
    o_ref[...] = jnp.maximum(
        jnp.dot(p_ref[...], w_ref[...], preferred_element_type=jnp.float32), 0.0
    )


_MM_BLOCK = 2000


def _mm_relu(p, w):
    return pl.pallas_call(
        _mm_body,
        grid=(N // _MM_BLOCK,),
        in_specs=[
            pl.BlockSpec((_MM_BLOCK, D_IN), lambda i: (i, 0)),
            pl.BlockSpec((D_IN, D_OUT), lambda i: (0, 0)),
        ],
        out_specs=pl.BlockSpec((_MM_BLOCK, D_OUT), lambda i: (i, 0)),
        out_shape=jax.ShapeDtypeStruct((N, D_OUT), jnp.float32),
    )(p, w)


def kernel(adj_edge_index, inputs, W):
    row_t = adj_edge_index[0].reshape(NS, EPT_REAL)
    col_t = adj_edge_index[1].reshape(NS, EPT_REAL)
    pad_r = N + (jnp.arange(PAD, dtype=jnp.int32) % (N_PAD - N))
    pad_c = (jnp.arange(PAD, dtype=jnp.int32) * 41) % N
    row_p = jnp.concatenate(
        [row_t, jnp.broadcast_to(pad_r, (NS, PAD))], axis=1).reshape(NS, NB, K)
    col_p = jnp.concatenate(
        [col_t, jnp.broadcast_to(pad_c, (NS, PAD))], axis=1).reshape(NS, NB, K)
    x_r = inputs.reshape(2 * N, DH)
    agg = _sc_segment_sum(row_p, col_p, x_r)
    return _mm_relu(agg, W)
